# baseline (speedup 1.0000x reference)
import jax
import jax.numpy as jnp
from jax.experimental import pallas as pl
from jax.experimental.pallas import tpu as pltpu

_KC = 2048
_NB = 2


def _nnue_kernel(white_ref, black_ref, stm_ref, ftw_ref, ftb_ref,
                 l1w_ref, l1b_ref, l2w_ref, l2b_ref,
                 out_ref, acc_w, acc_b):
    k = pl.program_id(0)
    b = pl.program_id(1)
    nk = pl.num_programs(0)
    m = ftw_ref.shape[0]

    wblk = white_ref[...].astype(jnp.bfloat16)
    bblk = black_ref[...].astype(jnp.bfloat16)
    fblk = ftw_ref[...].astype(jnp.bfloat16)
    dn = (((1,), (1,)), ((), ()))
    pw = jax.lax.dot_general(wblk, fblk, dn, preferred_element_type=jnp.float32)
    pb = jax.lax.dot_general(bblk, fblk, dn, preferred_element_type=jnp.float32)

    @pl.when(k == 0)
    def _():
        acc_w[b] = pw
        acc_b[b] = pb

    @pl.when(k > 0)
    def _():
        acc_w[b] += pw
        acc_b[b] += pb

    @pl.when(k == nk - 1)
    def _():
        w = acc_w[b] + ftb_ref[...]
        bb = acc_b[b] + ftb_ref[...]
        stm = stm_ref[...]
        d = w - bb
        x1 = jnp.clip(bb + stm * d, 0.0, 1.0)
        x2 = jnp.clip(w - stm * d, 0.0, 1.0)
        h = jax.lax.dot_general(x1.astype(jnp.bfloat16),
                                l1w_ref[:, :m].astype(jnp.bfloat16), dn,
                                preferred_element_type=jnp.float32)
        h = h + jax.lax.dot_general(x2.astype(jnp.bfloat16),
                                    l1w_ref[:, m:].astype(jnp.bfloat16), dn,
                                    preferred_element_type=jnp.float32)
        h = jnp.clip(h + l1b_ref[...], 0.0, 1.0)
        out = jnp.sum(h * l2w_ref[...], axis=1, keepdims=True)
        out_ref[...] = out + l2b_ref[0, 0]


def kernel(white_features, black_features, stm, ft_w, ft_b, l1_w, l1_b, l2_w, l2_b):
    bsz, nfeat = white_features.shape
    m = ft_w.shape[0]
    n = l1_w.shape[0]
    nk = nfeat // _KC
    bt = bsz // _NB

    return pl.pallas_call(
        _nnue_kernel,
        grid=(nk, _NB),
        in_specs=[
            pl.BlockSpec((bt, _KC), lambda k, b: (b, k)),
            pl.BlockSpec((bt, _KC), lambda k, b: (b, k)),
            pl.BlockSpec((bt, 1), lambda k, b: (b, 0)),
            pl.BlockSpec((m, _KC), lambda k, b: (0, k)),
            pl.BlockSpec((1, m), lambda k, b: (0, 0)),
            pl.BlockSpec((n, 2 * m), lambda k, b: (0, 0)),
            pl.BlockSpec((1, n), lambda k, b: (0, 0)),
            pl.BlockSpec((1, n), lambda k, b: (0, 0)),
            pl.BlockSpec(memory_space=pltpu.SMEM),
        ],
        out_specs=pl.BlockSpec((bt, 1), lambda k, b: (b, 0)),
        out_shape=jax.ShapeDtypeStruct((bsz, 1), jnp.float32),
        scratch_shapes=[
            pltpu.VMEM((_NB, bt, m), jnp.float32),
            pltpu.VMEM((_NB, bt, m), jnp.float32),
        ],
        compiler_params=pltpu.CompilerParams(
            dimension_semantics=("arbitrary", "arbitrary")),
    )(white_features, black_features, stm, ft_w, ft_b.reshape(1, m),
      l1_w, l1_b.reshape(1, n), l2_w, l2_b.reshape(1, 1))


# speedup vs baseline: 1.0742x; 1.0742x over previous
import jax
import jax.numpy as jnp
from jax.experimental import pallas as pl
from jax.experimental.pallas import tpu as pltpu


def _nnue_kernel(white_ref, black_ref, stm_ref, ftw_ref, ftb_ref,
                 l1w_ref, l1b_ref, l2w_ref, l2b_ref,
                 out_ref, acc_w, acc_b):
    k = pl.program_id(0)
    nk = pl.num_programs(0)
    m = ftw_ref.shape[0]

    wblk = white_ref[...].astype(jnp.bfloat16)
    bblk = black_ref[...].astype(jnp.bfloat16)
    fblk = ftw_ref[...].astype(jnp.bfloat16)
    dn = (((1,), (1,)), ((), ()))
    pw = jax.lax.dot_general(wblk, fblk, dn, preferred_element_type=jnp.float32)
    pb = jax.lax.dot_general(bblk, fblk, dn, preferred_element_type=jnp.float32)

    @pl.when(k == 0)
    def _():
        acc_w[...] = pw
        acc_b[...] = pb

    @pl.when(k > 0)
    def _():
        acc_w[...] += pw
        acc_b[...] += pb

    @pl.when(k == nk - 1)
    def _():
        w = acc_w[...] + ftb_ref[...]
        b = acc_b[...] + ftb_ref[...]
        stm = stm_ref[...]
        d = w - b
        x1 = jnp.clip(b + stm * d, 0.0, 1.0)
        x2 = jnp.clip(w - stm * d, 0.0, 1.0)
        h = jax.lax.dot_general(x1.astype(jnp.bfloat16),
                                l1w_ref[:, :m].astype(jnp.bfloat16), dn,
                                preferred_element_type=jnp.float32)
        h = h + jax.lax.dot_general(x2.astype(jnp.bfloat16),
                                    l1w_ref[:, m:].astype(jnp.bfloat16), dn,
                                    preferred_element_type=jnp.float32)
        h = jnp.clip(h + l1b_ref[...], 0.0, 1.0)
        out = jnp.sum(h * l2w_ref[...], axis=1, keepdims=True)
        out_ref[...] = out + l2b_ref[0, 0]


def kernel(white_features, black_features, stm, ft_w, ft_b, l1_w, l1_b, l2_w, l2_b):
    bsz, nfeat = white_features.shape
    m = ft_w.shape[0]
    n = l1_w.shape[0]
    kc = 2048
    nk = nfeat // kc

    return pl.pallas_call(
        _nnue_kernel,
        grid=(nk,),
        in_specs=[
            pl.BlockSpec((bsz, kc), lambda k: (0, k)),
            pl.BlockSpec((bsz, kc), lambda k: (0, k)),
            pl.BlockSpec((bsz, 1), lambda k: (0, 0)),
            pl.BlockSpec((m, kc), lambda k: (0, k)),
            pl.BlockSpec((1, m), lambda k: (0, 0)),
            pl.BlockSpec((n, 2 * m), lambda k: (0, 0)),
            pl.BlockSpec((1, n), lambda k: (0, 0)),
            pl.BlockSpec((1, n), lambda k: (0, 0)),
            pl.BlockSpec(memory_space=pltpu.SMEM),
        ],
        out_specs=pl.BlockSpec((bsz, 1), lambda k: (0, 0)),
        out_shape=jax.ShapeDtypeStruct((bsz, 1), jnp.float32),
        scratch_shapes=[
            pltpu.VMEM((bsz, m), jnp.float32),
            pltpu.VMEM((bsz, m), jnp.float32),
        ],
        compiler_params=pltpu.CompilerParams(dimension_semantics=("arbitrary",)),
    )(white_features, black_features, stm, ft_w, ft_b.reshape(1, m),
      l1_w, l1_b.reshape(1, n), l2_w, l2_b.reshape(1, 1))
